# Initial kernel scaffold; baseline (speedup 1.0000x reference)
import functools
import jax
import jax.numpy as jnp
from jax import lax
from jax.experimental import pallas as pl
from jax.experimental.pallas import tpu as pltpu
from jax.experimental.pallas import tpu_sc as plsc

D = 128
B_TOTAL = 4096 * 50
NW = 32
B_PER_W = B_TOTAL // NW
CHUNK = 128
N_CHUNKS = B_PER_W // CHUNK

_mesh = plsc.VectorSubcoreMesh(core_axis_name="c", subcore_axis_name="s")


@functools.partial(
    pl.kernel,
    mesh=_mesh,
    out_type=jax.ShapeDtypeStruct((B_TOTAL, D), jnp.float32),
    scratch_types=[
        pltpu.VMEM((N_CHUNKS, CHUNK), jnp.int32),
        pltpu.VMEM((CHUNK, D), jnp.float32),
        pltpu.SemaphoreType.DMA,
    ],
)
def _gather_kernel(idx_hbm, table_hbm, out_hbm, idx_v, rows_v, sem):
    wid = lax.axis_index("s") * 2 + lax.axis_index("c")
    base = wid * N_CHUNKS
    pltpu.sync_copy(idx_hbm.at[pl.ds(base, N_CHUNKS)], idx_v)

    def body(j, carry):
        pltpu.async_copy(table_hbm.at[idx_v.at[j]], rows_v, sem).wait()
        pltpu.sync_copy(rows_v, out_hbm.at[pl.ds((base + j) * CHUNK, CHUNK)])
        return carry

    lax.fori_loop(0, N_CHUNKS, body, 0)


def kernel(x, target, emb_table):
    idx = target.astype(jnp.int32).reshape(B_TOTAL // CHUNK, CHUNK)
    out = _gather_kernel(idx, emb_table)
    return out.reshape(target.shape[0], target.shape[1], D)


# speedup vs baseline: 2.9768x; 2.9768x over previous
import functools
import jax
import jax.numpy as jnp
from jax import lax
from jax.experimental import pallas as pl
from jax.experimental.pallas import tpu as pltpu
from jax.experimental.pallas import tpu_sc as plsc

D = 128
B_TOTAL = 4096 * 50
NW = 32
B_PER_W = B_TOTAL // NW
CHUNK = 128
N_CHUNKS = B_PER_W // CHUNK

_mesh = plsc.VectorSubcoreMesh(core_axis_name="c", subcore_axis_name="s")


@functools.partial(
    pl.kernel,
    mesh=_mesh,
    out_type=jax.ShapeDtypeStruct((B_TOTAL, D), jnp.float32),
    scratch_types=[
        pltpu.VMEM((N_CHUNKS, CHUNK), jnp.int32),
        pltpu.VMEM((CHUNK, D), jnp.float32),
        pltpu.SemaphoreType.DMA,
    ],
)
def _gather_kernel(idx_hbm, table_hbm, out_hbm, idx_v, rows_v, sem):
    wid = lax.axis_index("s") * 2 + lax.axis_index("c")
    base = wid * B_PER_W
    pltpu.sync_copy(idx_hbm.at[wid], idx_v)

    def body(j, carry):
        pltpu.async_copy(table_hbm.at[idx_v.at[j]], rows_v, sem).wait()
        pltpu.sync_copy(rows_v, out_hbm.at[pl.ds(base + j * CHUNK, CHUNK)])
        return carry

    lax.fori_loop(0, N_CHUNKS, body, 0)


def kernel(x, target, emb_table):
    idx = target.astype(jnp.int32).reshape(NW, N_CHUNKS, CHUNK)
    out = _gather_kernel(idx, emb_table)
    return out.reshape(target.shape[0], target.shape[1], D)


# speedup vs baseline: 3.3427x; 1.1229x over previous
import functools
import jax
import jax.numpy as jnp
from jax import lax
from jax.experimental import pallas as pl
from jax.experimental.pallas import tpu as pltpu
from jax.experimental.pallas import tpu_sc as plsc

D = 128
B_TOTAL = 4096 * 50
NW = 32
B_PER_W = B_TOTAL // NW
CHUNK = 128
N_CHUNKS = B_PER_W // CHUNK
NBUF = 5

_mesh = plsc.VectorSubcoreMesh(core_axis_name="c", subcore_axis_name="s")


@functools.partial(
    pl.kernel,
    mesh=_mesh,
    out_type=jax.ShapeDtypeStruct((B_TOTAL, D), jnp.float32),
    scratch_types=[
        pltpu.VMEM((N_CHUNKS, CHUNK), jnp.int32),
        [pltpu.VMEM((CHUNK, D), jnp.float32) for _ in range(NBUF)],
        [pltpu.SemaphoreType.DMA for _ in range(NBUF)],
        [pltpu.SemaphoreType.DMA for _ in range(NBUF)],
    ],
)
def _gather_kernel(idx_hbm, table_hbm, out_hbm, idx_v, bufs, gsems, wsems):
    wid = lax.axis_index("s") * 2 + lax.axis_index("c")
    base = wid * B_PER_W
    pltpu.sync_copy(idx_hbm.at[wid], idx_v)

    def gather(j, b):
        pltpu.async_copy(table_hbm.at[idx_v.at[j]], bufs[b], gsems[b])

    def write(j, b):
        pltpu.async_copy(
            bufs[b], out_hbm.at[pl.ds(base + j * CHUNK, CHUNK)], wsems[b]
        )

    for b in range(NBUF):
        gather(b, b)

    @pl.loop(0, N_CHUNKS - NBUF, step=NBUF)
    def _main(g0):
        for b in range(NBUF):
            j = g0 + b
            pltpu.make_async_copy(table_hbm.at[idx_v.at[j]], bufs[b], gsems[b]).wait()
            write(j, b)
            pltpu.make_async_copy(
                bufs[b], out_hbm.at[pl.ds(base + j * CHUNK, CHUNK)], wsems[b]
            ).wait()
            gather(j + NBUF, b)

    for b in range(NBUF):
        j = N_CHUNKS - NBUF + b
        pltpu.make_async_copy(table_hbm.at[idx_v.at[j]], bufs[b], gsems[b]).wait()
        write(j, b)
    for b in range(NBUF):
        j = N_CHUNKS - NBUF + b
        pltpu.make_async_copy(
            bufs[b], out_hbm.at[pl.ds(base + j * CHUNK, CHUNK)], wsems[b]
        ).wait()


def kernel(x, target, emb_table):
    idx = target.astype(jnp.int32).reshape(NW, N_CHUNKS, CHUNK)
    out = _gather_kernel(idx, emb_table)
    return out.reshape(target.shape[0], target.shape[1], D)


# speedup vs baseline: 3.3469x; 1.0013x over previous
import functools
import jax
import jax.numpy as jnp
from jax import lax
from jax.experimental import pallas as pl
from jax.experimental.pallas import tpu as pltpu
from jax.experimental.pallas import tpu_sc as plsc

D = 128
B_TOTAL = 4096 * 50
NW = 32
B_PER_W = B_TOTAL // NW
CHUNK = 128
N_CHUNKS = B_PER_W // CHUNK
NBUF = 5
LOOKAHEAD = 3

_mesh = plsc.VectorSubcoreMesh(core_axis_name="c", subcore_axis_name="s")


@functools.partial(
    pl.kernel,
    mesh=_mesh,
    out_type=jax.ShapeDtypeStruct((B_TOTAL, D), jnp.float32),
    scratch_types=[
        pltpu.VMEM((N_CHUNKS, CHUNK), jnp.int32),
        [pltpu.VMEM((CHUNK, D), jnp.float32) for _ in range(NBUF)],
        [pltpu.SemaphoreType.DMA for _ in range(NBUF)],
        [pltpu.SemaphoreType.DMA for _ in range(NBUF)],
    ],
)
def _gather_kernel(idx_hbm, table_hbm, out_hbm, idx_v, bufs, gsems, wsems):
    wid = lax.axis_index("s") * 2 + lax.axis_index("c")
    base = wid * B_PER_W
    pltpu.sync_copy(idx_hbm.at[wid], idx_v)

    def gather(j, b):
        pltpu.async_copy(table_hbm.at[idx_v.at[j]], bufs[b], gsems[b])

    def gather_wait(j, b):
        pltpu.make_async_copy(table_hbm.at[idx_v.at[j]], bufs[b], gsems[b]).wait()

    def write(j, b):
        pltpu.async_copy(
            bufs[b], out_hbm.at[pl.ds(base + j * CHUNK, CHUNK)], wsems[b]
        )

    def write_wait(j, b):
        pltpu.make_async_copy(
            bufs[b], out_hbm.at[pl.ds(base + j * CHUNK, CHUNK)], wsems[b]
        ).wait()

    for j in range(LOOKAHEAD):
        gather(j, j % NBUF)
    for j in range(NBUF - LOOKAHEAD):
        gather(j + LOOKAHEAD, (j + LOOKAHEAD) % NBUF)
        gather_wait(j, j % NBUF)
        write(j, j % NBUF)

    start = NBUF - LOOKAHEAD
    n_main = N_CHUNKS - LOOKAHEAD - start
    assert n_main % NBUF == 0

    @pl.loop(start, start + n_main, step=NBUF)
    def _main(j0):
        for d in range(NBUF):
            j = j0 + d
            bg = (start + d + LOOKAHEAD) % NBUF
            write_wait(j + LOOKAHEAD - NBUF, bg)
            gather(j + LOOKAHEAD, bg)
            bw = (start + d) % NBUF
            gather_wait(j, bw)
            write(j, bw)

    for j in range(N_CHUNKS - LOOKAHEAD, N_CHUNKS):
        gather_wait(j, j % NBUF)
        write(j, j % NBUF)

    for j in range(N_CHUNKS - NBUF, N_CHUNKS):
        write_wait(j, j % NBUF)


def kernel(x, target, emb_table):
    idx = target.astype(jnp.int32).reshape(NW, N_CHUNKS, CHUNK)
    out = _gather_kernel(idx, emb_table)
    return out.reshape(target.shape[0], target.shape[1], D)


# speedup vs baseline: 5.9605x; 1.7809x over previous
import functools
import jax
import jax.numpy as jnp
from jax import lax
from jax.experimental import pallas as pl
from jax.experimental.pallas import tpu as pltpu
from jax.experimental.pallas import tpu_sc as plsc

BATCH = 4096
HIST = 50
D = 128
NW = 32
BATCH_PER_W = BATCH // NW
NBUF = 8
LOOKAHEAD = 4

_mesh = plsc.VectorSubcoreMesh(core_axis_name="c", subcore_axis_name="s")


@functools.partial(
    pl.kernel,
    mesh=_mesh,
    out_type=jax.ShapeDtypeStruct((BATCH, HIST, D), jnp.float32),
    scratch_types=[
        pltpu.VMEM((BATCH_PER_W, HIST), jnp.int32),
        [pltpu.VMEM((HIST, D), jnp.float32) for _ in range(NBUF)],
        [pltpu.SemaphoreType.DMA for _ in range(NBUF)],
        [pltpu.SemaphoreType.DMA for _ in range(NBUF)],
    ],
)
def _gather_kernel(idx_hbm, table_hbm, out_hbm, idx_v, bufs, gsems, wsems):
    wid = lax.axis_index("s") * 2 + lax.axis_index("c")
    base = wid * BATCH_PER_W
    pltpu.sync_copy(idx_hbm.at[wid], idx_v)

    def gather(j, b):
        pltpu.async_copy(table_hbm.at[idx_v.at[j]], bufs[b], gsems[b])

    def gather_wait(j, b):
        pltpu.make_async_copy(table_hbm.at[idx_v.at[j]], bufs[b], gsems[b]).wait()

    def write(j, b):
        pltpu.async_copy(bufs[b], out_hbm.at[base + j], wsems[b])

    def write_wait(j, b):
        pltpu.make_async_copy(bufs[b], out_hbm.at[base + j], wsems[b]).wait()

    for j in range(LOOKAHEAD):
        gather(j, j % NBUF)
    for j in range(NBUF - LOOKAHEAD):
        gather(j + LOOKAHEAD, (j + LOOKAHEAD) % NBUF)
        gather_wait(j, j % NBUF)
        write(j, j % NBUF)

    start = NBUF - LOOKAHEAD
    n_main = BATCH_PER_W - LOOKAHEAD - start
    assert n_main % NBUF == 0

    @pl.loop(start, start + n_main, step=NBUF)
    def _main(j0):
        for d in range(NBUF):
            j = j0 + d
            bg = (start + d + LOOKAHEAD) % NBUF
            write_wait(j + LOOKAHEAD - NBUF, bg)
            gather(j + LOOKAHEAD, bg)
            bw = (start + d) % NBUF
            gather_wait(j, bw)
            write(j, bw)

    for j in range(BATCH_PER_W - LOOKAHEAD, BATCH_PER_W):
        gather_wait(j, j % NBUF)
        write(j, j % NBUF)

    for j in range(BATCH_PER_W - NBUF, BATCH_PER_W):
        write_wait(j, j % NBUF)


def kernel(x, target, emb_table):
    idx = target.astype(jnp.int32).reshape(NW, BATCH_PER_W, HIST)
    return _gather_kernel(idx, emb_table)


# speedup vs baseline: 10.7479x; 1.8032x over previous
import functools
import jax
import jax.numpy as jnp
from jax import lax
from jax.experimental import pallas as pl
from jax.experimental.pallas import tpu as pltpu
from jax.experimental.pallas import tpu_sc as plsc

BATCH = 4096
HIST = 50
D = 128
NW = 32
COLS = BATCH // NW
NBUF = 5
LOOKAHEAD = 3

_mesh = plsc.VectorSubcoreMesh(core_axis_name="c", subcore_axis_name="s")


@functools.partial(
    pl.kernel,
    mesh=_mesh,
    out_type=jax.ShapeDtypeStruct((HIST, BATCH, D), jnp.float32),
    scratch_types=[
        pltpu.VMEM((HIST, COLS), jnp.int32),
        [pltpu.VMEM((COLS, D), jnp.float32) for _ in range(NBUF)],
        [pltpu.SemaphoreType.DMA for _ in range(NBUF)],
        [pltpu.SemaphoreType.DMA for _ in range(NBUF)],
    ],
)
def _gather_kernel(idx_hbm, table_hbm, out_hbm, idx_v, bufs, gsems, wsems):
    wid = lax.axis_index("s") * 2 + lax.axis_index("c")
    col0 = wid * COLS
    pltpu.sync_copy(idx_hbm.at[:, wid], idx_v)

    def gather(l, b):
        pltpu.async_copy(table_hbm.at[idx_v.at[l]], bufs[b], gsems[b])

    def gather_wait(l, b):
        pltpu.make_async_copy(table_hbm.at[idx_v.at[l]], bufs[b], gsems[b]).wait()

    def write(l, b):
        pltpu.async_copy(bufs[b], out_hbm.at[l, pl.ds(col0, COLS)], wsems[b])

    def write_wait(l, b):
        pltpu.make_async_copy(
            bufs[b], out_hbm.at[l, pl.ds(col0, COLS)], wsems[b]
        ).wait()

    for l in range(LOOKAHEAD):
        gather(l, l % NBUF)
    for l in range(NBUF - LOOKAHEAD):
        gather(l + LOOKAHEAD, (l + LOOKAHEAD) % NBUF)
        gather_wait(l, l % NBUF)
        write(l, l % NBUF)

    start = NBUF - LOOKAHEAD
    n_main = HIST - LOOKAHEAD - start
    assert n_main % NBUF == 0

    @pl.loop(start, start + n_main, step=NBUF)
    def _main(l0):
        for d in range(NBUF):
            l = l0 + d
            bg = (start + d + LOOKAHEAD) % NBUF
            write_wait(l + LOOKAHEAD - NBUF, bg)
            gather(l + LOOKAHEAD, bg)
            bw = (start + d) % NBUF
            gather_wait(l, bw)
            write(l, bw)

    for l in range(HIST - LOOKAHEAD, HIST):
        gather_wait(l, l % NBUF)
        write(l, l % NBUF)

    for l in range(HIST - NBUF, HIST):
        write_wait(l, l % NBUF)


def kernel(x, target, emb_table):
    idx = jnp.transpose(target.astype(jnp.int32)).reshape(HIST, NW, COLS)
    out_t = _gather_kernel(idx, emb_table)
    return jnp.transpose(out_t, (1, 0, 2))


# speedup vs baseline: 10.7930x; 1.0042x over previous
import functools
import jax
import jax.numpy as jnp
from jax import lax
from jax.experimental import pallas as pl
from jax.experimental.pallas import tpu as pltpu
from jax.experimental.pallas import tpu_sc as plsc

BATCH = 4096
HIST = 50
D = 128
NW = 32
COLS = BATCH // NW
NBUF = 5
LOOKAHEAD = 4

_mesh = plsc.VectorSubcoreMesh(core_axis_name="c", subcore_axis_name="s")


@functools.partial(
    pl.kernel,
    mesh=_mesh,
    out_type=jax.ShapeDtypeStruct((HIST, BATCH, D), jnp.float32),
    scratch_types=[
        pltpu.VMEM((HIST, COLS), jnp.int32),
        [pltpu.VMEM((COLS, D), jnp.float32) for _ in range(NBUF)],
        [pltpu.SemaphoreType.DMA for _ in range(NBUF)],
        [pltpu.SemaphoreType.DMA for _ in range(NBUF)],
    ],
)
def _gather_kernel(idx_hbm, table_hbm, out_hbm, idx_v, bufs, gsems, wsems):
    wid = lax.axis_index("s") * 2 + lax.axis_index("c")
    col0 = wid * COLS
    pltpu.sync_copy(idx_hbm.at[:, wid], idx_v)

    def gather(l, b):
        pltpu.async_copy(table_hbm.at[idx_v.at[l]], bufs[b], gsems[b])

    def gather_wait(l, b):
        pltpu.make_async_copy(table_hbm.at[idx_v.at[l]], bufs[b], gsems[b]).wait()

    def write(l, b):
        pltpu.async_copy(bufs[b], out_hbm.at[l, pl.ds(col0, COLS)], wsems[b])

    def write_wait(l, b):
        pltpu.make_async_copy(
            bufs[b], out_hbm.at[l, pl.ds(col0, COLS)], wsems[b]
        ).wait()

    for l in range(LOOKAHEAD):
        gather(l, l % NBUF)
    for l in range(NBUF - LOOKAHEAD):
        gather(l + LOOKAHEAD, (l + LOOKAHEAD) % NBUF)
        gather_wait(l, l % NBUF)
        write(l, l % NBUF)

    start = NBUF - LOOKAHEAD
    n_main = HIST - LOOKAHEAD - start
    assert n_main % NBUF == 0

    @pl.loop(start, start + n_main, step=NBUF)
    def _main(l0):
        for d in range(NBUF):
            l = l0 + d
            bg = (start + d + LOOKAHEAD) % NBUF
            write_wait(l + LOOKAHEAD - NBUF, bg)
            gather(l + LOOKAHEAD, bg)
            bw = (start + d) % NBUF
            gather_wait(l, bw)
            write(l, bw)

    for l in range(HIST - LOOKAHEAD, HIST):
        gather_wait(l, l % NBUF)
        write(l, l % NBUF)

    for l in range(HIST - NBUF, HIST):
        write_wait(l, l % NBUF)


def kernel(x, target, emb_table):
    idx = jnp.transpose(target.astype(jnp.int32)).reshape(HIST, NW, COLS)
    out_t = _gather_kernel(idx, emb_table)
    return jnp.transpose(out_t, (1, 0, 2))
